# baseline (speedup 1.0000x reference)
import functools

import jax
import jax.numpy as jnp
from jax import lax
from jax.experimental import pallas as pl
from jax.experimental.pallas import tpu as pltpu
from jax.experimental.pallas import tpu_sc as plsc

NC = 2
NS = 16
NW = NC * NS
L = 16
K = 256

UNROLL = 8
INNER_UNROLL = 2
STEPS = (128, 64, 32, 16, 8, 4, 2, 1)
ROW_TILE = 8


def _search_group(midrep_v, svalrep_v, xin_v, xout_v, r, off, inv, pinit):
    xs = [xin_v[r, pl.ds(off + u * L, L)] * inv for u in range(UNROLL)]
    p = [pinit for _ in range(UNROLL)]
    for s in STEPS[:-1]:
        probes = [plsc.load_gather(midrep_v, [p[u]]) for u in range(UNROLL)]
        h = (s // 2) * L
        for u in range(UNROLL):
            p[u] = p[u] + jnp.where(probes[u] < xs[u], h, -h)
    probes = [plsc.load_gather(midrep_v, [p[u]]) for u in range(UNROLL)]
    idx = [p[u] + jnp.where(probes[u] < xs[u], L, 0) for u in range(UNROLL)]
    q = [plsc.load_gather(svalrep_v, [idx[u]]) for u in range(UNROLL)]
    for u in range(UNROLL):
        xout_v[r, pl.ds(off + u * L, L)] = q[u]


def _make_sc_kernel(rows, cols, per_w_rows, chunk_rows):
    n_chunks = per_w_rows // chunk_rows
    n_pairs = n_chunks // 2

    mesh = plsc.VectorSubcoreMesh(core_axis_name="c", subcore_axis_name="s")

    @functools.partial(
        pl.kernel,
        out_type=jax.ShapeDtypeStruct((rows, cols), jnp.float32),
        mesh=mesh,
        compiler_params=pltpu.CompilerParams(
            needs_layout_passes=False, use_tc_tiling_on_sc=True),
        scratch_types=[
            pltpu.VMEM((K,), jnp.float32),
            pltpu.VMEM((K,), jnp.float32),
            pltpu.VMEM((K,), jnp.float32),
            pltpu.VMEM((K * L,), jnp.float32),
            pltpu.VMEM((K * L,), jnp.float32),
            pltpu.VMEM((L,), jnp.float32),
            pltpu.VMEM((L,), jnp.float32),
            pltpu.VMEM((chunk_rows, cols), jnp.float32),
            pltpu.VMEM((chunk_rows, cols), jnp.float32),
            pltpu.VMEM((chunk_rows, cols), jnp.float32),
            pltpu.VMEM((chunk_rows, cols), jnp.float32),
            pltpu.SemaphoreType.DMA,
            pltpu.SemaphoreType.DMA,
            pltpu.SemaphoreType.DMA,
            pltpu.SemaphoreType.DMA,
        ],
    )
    def sc_kernel(x_hbm, grid_hbm, alpha_hbm, inv_hbm, out_hbm,
                  grid_v, mid_v, sval_v, midrep_v, svalrep_v,
                  a_v, i_v, in0_v, in1_v, out0_v, out1_v,
                  si0, si1, so0, so1):
        wid = lax.axis_index("s") * NC + lax.axis_index("c")
        base = wid * per_w_rows

        pltpu.sync_copy(grid_hbm, grid_v)
        pltpu.sync_copy(alpha_hbm, a_v)
        pltpu.sync_copy(inv_hbm, i_v)

        alpha = a_v[...]
        inv = i_v[...]

        for i in range(K // L):
            lo = grid_v[pl.ds(i * L, L)]
            hi_idx = jnp.minimum(lax.iota(jnp.int32, L) + (i * L + 1), K - 1)
            hi = plsc.load_gather(grid_v, [hi_idx])
            mid_v[pl.ds(i * L, L)] = (lo + hi) * 0.5
            sval_v[pl.ds(i * L, L)] = lo * alpha

        lane = lax.iota(jnp.int32, L)

        @plsc.parallel_loop(0, K // L, 1)
        def replicate(i):
            mv = mid_v[pl.ds(i * L, L)]
            sv = sval_v[pl.ds(i * L, L)]
            base = lane * L + K * i
            for c in range(L):
                idxv = base + ((lane + c) & (L - 1))
                plsc.store_scatter(midrep_v, [idxv], mv)
                plsc.store_scatter(svalrep_v, [idxv], sv)
        pinit = lane + (STEPS[0] - 1) * L

        def compute(xin_v, xout_v):
            def row_body(r, carry):
                @plsc.parallel_loop(0, cols, L * UNROLL, unroll=INNER_UNROLL)
                def group_body(off):
                    off = pl.multiple_of(off, L * UNROLL)
                    _search_group(midrep_v, svalrep_v, xin_v, xout_v, r, off,
                                  inv, pinit)
                return carry

            lax.fori_loop(0, chunk_rows, row_body, 0, unroll=False)

        pltpu.async_copy(x_hbm.at[pl.ds(base, chunk_rows)], in0_v, si0)
        pltpu.async_copy(
            x_hbm.at[pl.ds(base + chunk_rows, chunk_rows)], in1_v, si1)

        def half(k, start, in_v, out_v, si, so):
            pltpu.make_async_copy(
                x_hbm.at[pl.ds(start, chunk_rows)], in_v, si).wait()

            @pl.when(k > 0)
            def _():
                pltpu.make_async_copy(
                    out_v, out_hbm.at[pl.ds(start - 2 * chunk_rows,
                                            chunk_rows)], so).wait()

            compute(in_v, out_v)
            pltpu.async_copy(out_v, out_hbm.at[pl.ds(start, chunk_rows)], so)

            @pl.when(k + 1 < n_pairs)
            def _():
                pltpu.async_copy(
                    x_hbm.at[pl.ds(start + 2 * chunk_rows, chunk_rows)],
                    in_v, si)

        def pair_body(k, carry):
            start0 = base + (2 * k) * chunk_rows
            half(k, start0, in0_v, out0_v, si0, so0)
            half(k, start0 + chunk_rows, in1_v, out1_v, si1, so1)
            return carry

        lax.fori_loop(0, n_pairs, pair_body, 0, unroll=False)
        end = base + n_chunks * chunk_rows
        pltpu.make_async_copy(
            out0_v, out_hbm.at[pl.ds(end - 2 * chunk_rows, chunk_rows)],
            so0).wait()
        pltpu.make_async_copy(
            out1_v, out_hbm.at[pl.ds(end - chunk_rows, chunk_rows)],
            so1).wait()

    return sc_kernel


@jax.jit
def kernel(x, quant_grid, alpha):
    cols = x.shape[-1]
    rows = x.size // cols
    assert cols % (L * UNROLL) == 0 and rows % (NW * 2 * ROW_TILE) == 0
    per_w_rows = rows // NW
    chunk_rows = None
    for c in range(16, 0, -1):
        if per_w_rows % c == 0 and (per_w_rows // c) % 2 == 0 \
                and c % ROW_TILE == 0:
            chunk_rows = c
            break
    assert chunk_rows is not None

    x2 = x.reshape(rows, cols).astype(jnp.float32)
    alpha_f = jnp.asarray(alpha, jnp.float32)
    a_vec = jnp.broadcast_to(alpha_f, (L,))
    i_vec = jnp.broadcast_to(1.0 / alpha_f, (L,))

    out = _make_sc_kernel(rows, cols, per_w_rows, chunk_rows)(
        x2, quant_grid.astype(jnp.float32), a_vec, i_vec)
    return out.reshape(x.shape)


# speedup vs baseline: 1.0252x; 1.0252x over previous
import functools

import jax
import jax.numpy as jnp
from jax import lax
from jax.experimental import pallas as pl
from jax.experimental.pallas import tpu as pltpu
from jax.experimental.pallas import tpu_sc as plsc

NC = 2
NS = 16
NW = NC * NS
L = 16
K = 256

UNROLL = 8
INNER_UNROLL = 1
STEPS = (128, 64, 32, 16, 8, 4, 2, 1)
ROW_TILE = 8


def _search_group(midrep_v, svalrep_v, xin_v, xout_v, r, off, inv, pinit):
    xs = [xin_v[r, pl.ds(off + u * L, L)] * inv for u in range(UNROLL)]
    p = [pinit for _ in range(UNROLL)]
    for s in STEPS[:-1]:
        probes = [plsc.load_gather(midrep_v, [p[u]]) for u in range(UNROLL)]
        h = (s // 2) * L
        for u in range(UNROLL):
            p[u] = p[u] + jnp.where(probes[u] < xs[u], h, -h)
    probes = [plsc.load_gather(midrep_v, [p[u]]) for u in range(UNROLL)]
    idx = [p[u] + jnp.where(probes[u] < xs[u], L, 0) for u in range(UNROLL)]
    q = [plsc.load_gather(svalrep_v, [idx[u]]) for u in range(UNROLL)]
    for u in range(UNROLL):
        xout_v[r, pl.ds(off + u * L, L)] = q[u]


def _make_sc_kernel(rows, cols, per_w_rows, chunk_rows):
    n_chunks = per_w_rows // chunk_rows
    n_pairs = n_chunks // 2

    mesh = plsc.VectorSubcoreMesh(core_axis_name="c", subcore_axis_name="s")

    @functools.partial(
        pl.kernel,
        out_type=jax.ShapeDtypeStruct((rows, cols), jnp.float32),
        mesh=mesh,
        compiler_params=pltpu.CompilerParams(
            needs_layout_passes=False, use_tc_tiling_on_sc=True),
        scratch_types=[
            pltpu.VMEM((K,), jnp.float32),
            pltpu.VMEM((K,), jnp.float32),
            pltpu.VMEM((K,), jnp.float32),
            pltpu.VMEM((K * L,), jnp.float32),
            pltpu.VMEM((K * L,), jnp.float32),
            pltpu.VMEM((L,), jnp.float32),
            pltpu.VMEM((L,), jnp.float32),
            pltpu.VMEM((chunk_rows, cols), jnp.float32),
            pltpu.VMEM((chunk_rows, cols), jnp.float32),
            pltpu.VMEM((chunk_rows, cols), jnp.float32),
            pltpu.VMEM((chunk_rows, cols), jnp.float32),
            pltpu.SemaphoreType.DMA,
            pltpu.SemaphoreType.DMA,
            pltpu.SemaphoreType.DMA,
            pltpu.SemaphoreType.DMA,
        ],
    )
    def sc_kernel(x_hbm, grid_hbm, alpha_hbm, inv_hbm, out_hbm,
                  grid_v, mid_v, sval_v, midrep_v, svalrep_v,
                  a_v, i_v, in0_v, in1_v, out0_v, out1_v,
                  si0, si1, so0, so1):
        wid = lax.axis_index("s") * NC + lax.axis_index("c")
        base = wid * per_w_rows

        pltpu.sync_copy(grid_hbm, grid_v)
        pltpu.sync_copy(alpha_hbm, a_v)
        pltpu.sync_copy(inv_hbm, i_v)

        alpha = a_v[...]
        inv = i_v[...]

        for i in range(K // L):
            lo = grid_v[pl.ds(i * L, L)]
            hi_idx = jnp.minimum(lax.iota(jnp.int32, L) + (i * L + 1), K - 1)
            hi = plsc.load_gather(grid_v, [hi_idx])
            mid_v[pl.ds(i * L, L)] = (lo + hi) * 0.5
            sval_v[pl.ds(i * L, L)] = lo * alpha

        @plsc.parallel_loop(0, K, 1, unroll=4)
        def replicate(j):
            jv = jnp.full((L,), 0, jnp.int32) + j
            midrep_v[pl.ds(j * L, L)] = plsc.load_gather(mid_v, [jv])
            svalrep_v[pl.ds(j * L, L)] = plsc.load_gather(sval_v, [jv])

        lane = lax.iota(jnp.int32, L)
        pinit = lane + (STEPS[0] - 1) * L

        def compute(xin_v, xout_v):
            def row_body(r, carry):
                @plsc.parallel_loop(0, cols, L * UNROLL, unroll=INNER_UNROLL)
                def group_body(off):
                    off = pl.multiple_of(off, L * UNROLL)
                    _search_group(midrep_v, svalrep_v, xin_v, xout_v, r, off,
                                  inv, pinit)
                return carry

            lax.fori_loop(0, chunk_rows, row_body, 0, unroll=False)

        pltpu.async_copy(x_hbm.at[pl.ds(base, chunk_rows)], in0_v, si0)
        pltpu.async_copy(
            x_hbm.at[pl.ds(base + chunk_rows, chunk_rows)], in1_v, si1)

        def half(k, start, in_v, out_v, si, so):
            pltpu.make_async_copy(
                x_hbm.at[pl.ds(start, chunk_rows)], in_v, si).wait()

            @pl.when(k > 0)
            def _():
                pltpu.make_async_copy(
                    out_v, out_hbm.at[pl.ds(start - 2 * chunk_rows,
                                            chunk_rows)], so).wait()

            compute(in_v, out_v)
            pltpu.async_copy(out_v, out_hbm.at[pl.ds(start, chunk_rows)], so)

            @pl.when(k + 1 < n_pairs)
            def _():
                pltpu.async_copy(
                    x_hbm.at[pl.ds(start + 2 * chunk_rows, chunk_rows)],
                    in_v, si)

        def pair_body(k, carry):
            start0 = base + (2 * k) * chunk_rows
            half(k, start0, in0_v, out0_v, si0, so0)
            half(k, start0 + chunk_rows, in1_v, out1_v, si1, so1)
            return carry

        lax.fori_loop(0, n_pairs, pair_body, 0, unroll=False)
        end = base + n_chunks * chunk_rows
        pltpu.make_async_copy(
            out0_v, out_hbm.at[pl.ds(end - 2 * chunk_rows, chunk_rows)],
            so0).wait()
        pltpu.make_async_copy(
            out1_v, out_hbm.at[pl.ds(end - chunk_rows, chunk_rows)],
            so1).wait()

    return sc_kernel


@jax.jit
def kernel(x, quant_grid, alpha):
    cols = x.shape[-1]
    rows = x.size // cols
    assert cols % (L * UNROLL) == 0 and rows % (NW * 2 * ROW_TILE) == 0
    per_w_rows = rows // NW
    chunk_rows = None
    for c in range(16, 0, -1):
        if per_w_rows % c == 0 and (per_w_rows // c) % 2 == 0 \
                and c % ROW_TILE == 0:
            chunk_rows = c
            break
    assert chunk_rows is not None

    x2 = x.reshape(rows, cols).astype(jnp.float32)
    alpha_f = jnp.asarray(alpha, jnp.float32)
    a_vec = jnp.broadcast_to(alpha_f, (L,))
    i_vec = jnp.broadcast_to(1.0 / alpha_f, (L,))

    out = _make_sc_kernel(rows, cols, per_w_rows, chunk_rows)(
        x2, quant_grid.astype(jnp.float32), a_vec, i_vec)
    return out.reshape(x.shape)


# speedup vs baseline: 1.1269x; 1.0992x over previous
import functools

import jax
import jax.numpy as jnp
from jax import lax
from jax.experimental import pallas as pl
from jax.experimental.pallas import tpu as pltpu
from jax.experimental.pallas import tpu_sc as plsc

NC = 2
NS = 16
NW = NC * NS
L = 16
K = 256

UNROLL = 8
INNER_UNROLL = 1
STEPS = (128, 64, 32, 16, 8, 4, 2, 1)
ROW_TILE = 8

_GATHER_DNUMS = lax.GatherDimensionNumbers(
    offset_dims=(), collapsed_slice_dims=(0,), start_index_map=(0,))


def _reg_gather(table, idx):
    return lax.gather(
        table, idx[:, None], _GATHER_DNUMS, (1,),
        mode=lax.GatherScatterMode.PROMISE_IN_BOUNDS)


def _search_group(midrep_v, svalrep_v, coarse, xin_v, xout_v, r, off, inv,
                  lane112):
    xs = [xin_v[r, pl.ds(off + u * L, L)] * inv for u in range(UNROLL)]

    c = [jnp.full((L,), 7, jnp.int32) for _ in range(UNROLL)]
    for s in (8, 4, 2):
        probes = [_reg_gather(coarse, c[u])
                  for u in range(UNROLL)]
        h = s // 2
        for u in range(UNROLL):
            c[u] = c[u] + jnp.where(probes[u] < xs[u], h, -h)
    probes = [_reg_gather(coarse, c[u])
              for u in range(UNROLL)]
    p = [((c[u] + jnp.where(probes[u] < xs[u], 1, 0)) << 8) + lane112
         for u in range(UNROLL)]

    for s in (8, 4, 2):
        probes = [plsc.load_gather(midrep_v, [p[u]]) for u in range(UNROLL)]
        h = (s // 2) * L
        for u in range(UNROLL):
            p[u] = p[u] + jnp.where(probes[u] < xs[u], h, -h)
    probes = [plsc.load_gather(midrep_v, [p[u]]) for u in range(UNROLL)]
    idx = [p[u] + jnp.where(probes[u] < xs[u], L, 0) for u in range(UNROLL)]
    q = [plsc.load_gather(svalrep_v, [idx[u]]) for u in range(UNROLL)]
    for u in range(UNROLL):
        xout_v[r, pl.ds(off + u * L, L)] = q[u]


def _make_sc_kernel(rows, cols, per_w_rows, chunk_rows):
    n_chunks = per_w_rows // chunk_rows
    n_pairs = n_chunks // 2

    mesh = plsc.VectorSubcoreMesh(core_axis_name="c", subcore_axis_name="s")

    @functools.partial(
        pl.kernel,
        out_type=jax.ShapeDtypeStruct((rows, cols), jnp.float32),
        mesh=mesh,
        compiler_params=pltpu.CompilerParams(
            needs_layout_passes=False, use_tc_tiling_on_sc=True),
        scratch_types=[
            pltpu.VMEM((K,), jnp.float32),
            pltpu.VMEM((K,), jnp.float32),
            pltpu.VMEM((K,), jnp.float32),
            pltpu.VMEM((K * L,), jnp.float32),
            pltpu.VMEM((K * L,), jnp.float32),
            pltpu.VMEM((L,), jnp.float32),
            pltpu.VMEM((L,), jnp.float32),
            pltpu.VMEM((chunk_rows, cols), jnp.float32),
            pltpu.VMEM((chunk_rows, cols), jnp.float32),
            pltpu.VMEM((chunk_rows, cols), jnp.float32),
            pltpu.VMEM((chunk_rows, cols), jnp.float32),
            pltpu.SemaphoreType.DMA,
            pltpu.SemaphoreType.DMA,
            pltpu.SemaphoreType.DMA,
            pltpu.SemaphoreType.DMA,
        ],
    )
    def sc_kernel(x_hbm, grid_hbm, alpha_hbm, inv_hbm, out_hbm,
                  grid_v, mid_v, sval_v, midrep_v, svalrep_v,
                  a_v, i_v, in0_v, in1_v, out0_v, out1_v,
                  si0, si1, so0, so1):
        wid = lax.axis_index("s") * NC + lax.axis_index("c")
        base = wid * per_w_rows

        pltpu.sync_copy(grid_hbm, grid_v)
        pltpu.sync_copy(alpha_hbm, a_v)
        pltpu.sync_copy(inv_hbm, i_v)

        alpha = a_v[...]
        inv = i_v[...]

        for i in range(K // L):
            lo = grid_v[pl.ds(i * L, L)]
            hi_idx = jnp.minimum(lax.iota(jnp.int32, L) + (i * L + 1), K - 1)
            hi = plsc.load_gather(grid_v, [hi_idx])
            mid_v[pl.ds(i * L, L)] = (lo + hi) * 0.5
            sval_v[pl.ds(i * L, L)] = lo * alpha

        @plsc.parallel_loop(0, K, 1, unroll=4)
        def replicate(j):
            jv = jnp.full((L,), 0, jnp.int32) + j
            midrep_v[pl.ds(j * L, L)] = plsc.load_gather(mid_v, [jv])
            svalrep_v[pl.ds(j * L, L)] = plsc.load_gather(sval_v, [jv])

        lane = lax.iota(jnp.int32, L)
        lane112 = lane + 7 * L
        coarse = plsc.load_gather(mid_v, [lane * L + (L - 1)])

        def compute(xin_v, xout_v):
            def row_body(r, carry):
                @plsc.parallel_loop(0, cols, L * UNROLL, unroll=INNER_UNROLL)
                def group_body(off):
                    off = pl.multiple_of(off, L * UNROLL)
                    _search_group(midrep_v, svalrep_v, coarse, xin_v, xout_v,
                                  r, off, inv, lane112)
                return carry

            lax.fori_loop(0, chunk_rows, row_body, 0, unroll=False)

        pltpu.async_copy(x_hbm.at[pl.ds(base, chunk_rows)], in0_v, si0)
        pltpu.async_copy(
            x_hbm.at[pl.ds(base + chunk_rows, chunk_rows)], in1_v, si1)

        def half(k, start, in_v, out_v, si, so):
            pltpu.make_async_copy(
                x_hbm.at[pl.ds(start, chunk_rows)], in_v, si).wait()

            @pl.when(k > 0)
            def _():
                pltpu.make_async_copy(
                    out_v, out_hbm.at[pl.ds(start - 2 * chunk_rows,
                                            chunk_rows)], so).wait()

            compute(in_v, out_v)
            pltpu.async_copy(out_v, out_hbm.at[pl.ds(start, chunk_rows)], so)

            @pl.when(k + 1 < n_pairs)
            def _():
                pltpu.async_copy(
                    x_hbm.at[pl.ds(start + 2 * chunk_rows, chunk_rows)],
                    in_v, si)

        def pair_body(k, carry):
            start0 = base + (2 * k) * chunk_rows
            half(k, start0, in0_v, out0_v, si0, so0)
            half(k, start0 + chunk_rows, in1_v, out1_v, si1, so1)
            return carry

        lax.fori_loop(0, n_pairs, pair_body, 0, unroll=False)
        end = base + n_chunks * chunk_rows
        pltpu.make_async_copy(
            out0_v, out_hbm.at[pl.ds(end - 2 * chunk_rows, chunk_rows)],
            so0).wait()
        pltpu.make_async_copy(
            out1_v, out_hbm.at[pl.ds(end - chunk_rows, chunk_rows)],
            so1).wait()

    return sc_kernel


@jax.jit
def kernel(x, quant_grid, alpha):
    cols = x.shape[-1]
    rows = x.size // cols
    assert cols % (L * UNROLL) == 0 and rows % (NW * 2 * ROW_TILE) == 0
    per_w_rows = rows // NW
    chunk_rows = None
    for c in range(16, 0, -1):
        if per_w_rows % c == 0 and (per_w_rows // c) % 2 == 0 \
                and c % ROW_TILE == 0:
            chunk_rows = c
            break
    assert chunk_rows is not None

    x2 = x.reshape(rows, cols).astype(jnp.float32)
    alpha_f = jnp.asarray(alpha, jnp.float32)
    a_vec = jnp.broadcast_to(alpha_f, (L,))
    i_vec = jnp.broadcast_to(1.0 / alpha_f, (L,))

    out = _make_sc_kernel(rows, cols, per_w_rows, chunk_rows)(
        x2, quant_grid.astype(jnp.float32), a_vec, i_vec)
    return out.reshape(x.shape)
